# Initial kernel scaffold; baseline (speedup 1.0000x reference)
import functools

import jax
import jax.numpy as jnp
from jax import lax
from jax.experimental import pallas as pl
from jax.experimental.pallas import tpu as pltpu
from jax.experimental.pallas import tpu_sc as plsc

N, E, D, DE = 10000, 320000, 128, 16
NC, NS, L = 2, 16, 16
NW = NC * NS
EW = E // NW
K = 80
NCHUNK = EW // K
RPT = N // NS
ZR = 25

_PREC = lax.Precision.HIGHEST

_mesh = plsc.VectorSubcoreMesh(
    core_axis_name="c", subcore_axis_name="s", num_cores=NC, num_subcores=NS)


@functools.partial(
    pl.kernel,
    out_type=jax.ShapeDtypeStruct((E, D), jnp.float32),
    mesh=_mesh,
    scratch_types=[
        pltpu.VMEM((K,), jnp.int32),
        pltpu.VMEM((K,), jnp.int32),
        pltpu.VMEM((K, D), jnp.float32),
        pltpu.VMEM((K, D), jnp.float32),
        pltpu.SemaphoreType.DMA,
    ],
)
def _gather_sum(a_hbm, b_hbm, dst_hbm, src_hbm, out_hbm, di, si, ga, gb, sem):
    wid = lax.axis_index("s") * NC + lax.axis_index("c")
    base = wid * EW

    @pl.loop(0, NCHUNK)
    def _chunk(c):
        off = base + c * K
        pltpu.sync_copy(dst_hbm.at[pl.ds(off, K)], di)
        pltpu.sync_copy(src_hbm.at[pl.ds(off, K)], si)
        cp_a = pltpu.async_copy(a_hbm.at[di], ga, sem)
        cp_b = pltpu.async_copy(b_hbm.at[si], gb, sem)
        cp_a.wait()
        cp_b.wait()

        @pl.loop(0, K)
        def _row(r):
            for j in range(D // L):
                sl = pl.ds(j * L, L)
                ga[r, sl] = ga[r, sl] + gb[r, sl]

        pltpu.sync_copy(ga, out_hbm.at[pl.ds(off, K)])


@functools.partial(
    pl.kernel,
    out_type=jax.ShapeDtypeStruct((NC, N, D), jnp.float32),
    mesh=_mesh,
    scratch_types=[
        pltpu.VMEM((K,), jnp.int32),
        pltpu.VMEM((K, D), jnp.float32),
        pltpu.VMEM((ZR, D), jnp.float32),
        pltpu.VMEM_SHARED((N, D), jnp.float32),
        pltpu.SemaphoreType.DMA,
    ],
)
def _scatter_add(m2_hbm, dst_hbm, out_hbm, idxb, mb, zb, aggr, sem):
    cid = lax.axis_index("c")
    sid = lax.axis_index("s")
    wid = sid * NC + cid
    base = wid * EW

    @pl.loop(0, ZR)
    def _zrow(r):
        for j in range(D // L):
            zb[r, pl.ds(j * L, L)] = jnp.zeros((L,), jnp.float32)

    @pl.loop(0, RPT // ZR)
    def _zcopy(z):
        pltpu.sync_copy(zb, aggr.at[pl.ds(sid * RPT + z * ZR, ZR)])

    plsc.subcore_barrier()

    @pl.loop(0, NCHUNK)
    def _chunk(c):
        off = base + c * K
        pltpu.sync_copy(dst_hbm.at[pl.ds(off, K)], idxb)
        pltpu.sync_copy(m2_hbm.at[pl.ds(off, K)], mb)
        pltpu.sync_copy(mb, aggr.at[idxb], add=True)

    plsc.subcore_barrier()

    pltpu.sync_copy(aggr.at[pl.ds(sid * RPT, RPT)],
                    out_hbm.at[cid].at[pl.ds(sid * RPT, RPT)])


def _dot(x, w):
    return jnp.dot(x, w, precision=_PREC, preferred_element_type=jnp.float32)


def _pre_body(h_ref, wa_ref, wb_ref, a_ref, b_ref):
    hblk = h_ref[...]
    a_ref[...] = _dot(hblk, wa_ref[...])
    b_ref[...] = _dot(hblk, wb_ref[...])


def _precompute(h, w1a, w1b):
    blk = 1000
    grid = N // blk
    return pl.pallas_call(
        _pre_body,
        grid=(grid,),
        in_specs=[
            pl.BlockSpec((blk, D), lambda i: (i, 0)),
            pl.BlockSpec((D, D), lambda i: (0, 0)),
            pl.BlockSpec((D, D), lambda i: (0, 0)),
        ],
        out_specs=[
            pl.BlockSpec((blk, D), lambda i: (i, 0)),
            pl.BlockSpec((blk, D), lambda i: (i, 0)),
        ],
        out_shape=[
            jax.ShapeDtypeStruct((N, D), jnp.float32),
            jax.ShapeDtypeStruct((N, D), jnp.float32),
        ],
        compiler_params=pltpu.CompilerParams(
            dimension_semantics=("arbitrary",)),
    )(h, w1a, w1b)


def _msg_body(g_ref, ea_ref, w1c_ref, b1_ref, w2_ref, b2_ref, o_ref):
    x = g_ref[...] + _dot(ea_ref[...], w1c_ref[...]) + b1_ref[...]
    x = jnp.maximum(x, 0.0)
    y = _dot(x, w2_ref[...]) + b2_ref[...]
    o_ref[...] = jnp.maximum(y, 0.0)


def _msg(g, ea, w1c, b1, w2, b2):
    blk = 2000
    grid = E // blk
    return pl.pallas_call(
        _msg_body,
        grid=(grid,),
        in_specs=[
            pl.BlockSpec((blk, D), lambda i: (i, 0)),
            pl.BlockSpec((blk, DE), lambda i: (i, 0)),
            pl.BlockSpec((DE, D), lambda i: (0, 0)),
            pl.BlockSpec((1, D), lambda i: (0, 0)),
            pl.BlockSpec((D, D), lambda i: (0, 0)),
            pl.BlockSpec((1, D), lambda i: (0, 0)),
        ],
        out_specs=pl.BlockSpec((blk, D), lambda i: (i, 0)),
        out_shape=jax.ShapeDtypeStruct((E, D), jnp.float32),
        compiler_params=pltpu.CompilerParams(
            dimension_semantics=("arbitrary",)),
    )(g, ea, w1c, b1, w2, b2)


def _upd_body(h_ref, p0_ref, p1_ref, ua_ref, ub_ref, bu1_ref, u2_ref,
              bu2_ref, o_ref):
    aggr = p0_ref[...] + p1_ref[...]
    u = _dot(h_ref[...], ua_ref[...]) + _dot(aggr, ub_ref[...]) + bu1_ref[...]
    u = jnp.maximum(u, 0.0)
    o_ref[...] = jnp.maximum(_dot(u, u2_ref[...]) + bu2_ref[...], 0.0)


def _update(h, p0, p1, ua, ub, bu1, u2, bu2):
    blk = 1000
    grid = N // blk
    return pl.pallas_call(
        _upd_body,
        grid=(grid,),
        in_specs=[
            pl.BlockSpec((blk, D), lambda i: (i, 0)),
            pl.BlockSpec((blk, D), lambda i: (i, 0)),
            pl.BlockSpec((blk, D), lambda i: (i, 0)),
            pl.BlockSpec((D, D), lambda i: (0, 0)),
            pl.BlockSpec((D, D), lambda i: (0, 0)),
            pl.BlockSpec((1, D), lambda i: (0, 0)),
            pl.BlockSpec((D, D), lambda i: (0, 0)),
            pl.BlockSpec((1, D), lambda i: (0, 0)),
        ],
        out_specs=pl.BlockSpec((blk, D), lambda i: (i, 0)),
        out_shape=jax.ShapeDtypeStruct((N, D), jnp.float32),
        compiler_params=pltpu.CompilerParams(
            dimension_semantics=("arbitrary",)),
    )(h, p0, p1, ua, ub, bu1, u2, bu2)


def kernel(h, edge_index, edge_attr, W1, b1, W2, b2, U1, bu1, U2, bu2):
    src = edge_index[0]
    dst = edge_index[1]
    w1a, w1b, w1c = W1[:D], W1[D:2 * D], W1[2 * D:]
    ua, ub = U1[:D], U1[D:]
    b1r = b1.reshape(1, D)
    b2r = b2.reshape(1, D)
    bu1r = bu1.reshape(1, D)
    bu2r = bu2.reshape(1, D)

    a, b = _precompute(h, w1a, w1b)
    g = _gather_sum(a, b, dst, src)
    m2 = _msg(g, edge_attr, w1c, b1r, W2, b2r)
    parts = _scatter_add(m2, dst)
    out = _update(h, parts[0], parts[1], ua, ub, bu1r, U2, bu2r)
    return out


# speedup vs baseline: 2.2183x; 2.2183x over previous
import functools

import jax
import jax.numpy as jnp
from jax import lax
from jax.experimental import pallas as pl
from jax.experimental.pallas import tpu as pltpu
from jax.experimental.pallas import tpu_sc as plsc

N, E, D, DE = 10000, 320000, 128, 16
NC, NS, L = 2, 16, 16
NW = NC * NS
EW = E // NW
K = 80
NCHUNK = EW // K
RPT = 624
REM = N - NS * RPT
ZR = 16

_PREC = lax.Precision.HIGHEST

_mesh = plsc.VectorSubcoreMesh(
    core_axis_name="c", subcore_axis_name="s", num_cores=NC, num_subcores=NS)


@functools.partial(
    pl.kernel,
    out_type=jax.ShapeDtypeStruct((E, D), jnp.float32),
    mesh=_mesh,
    scratch_types=[
        pltpu.VMEM((K,), jnp.int32),
        pltpu.VMEM((K,), jnp.int32),
        pltpu.VMEM((K, D), jnp.float32),
        pltpu.VMEM((K, D), jnp.float32),
        pltpu.SemaphoreType.DMA,
    ],
)
def _gather_sum(a_hbm, b_hbm, dst_hbm, src_hbm, out_hbm, di, si, ga, gb, sem):
    wid = lax.axis_index("s") * NC + lax.axis_index("c")
    base = wid * EW

    @pl.loop(0, NCHUNK)
    def _chunk(c):
        off = base + c * K
        pltpu.sync_copy(dst_hbm.at[pl.ds(off, K)], di)
        pltpu.sync_copy(src_hbm.at[pl.ds(off, K)], si)
        cp_a = pltpu.async_copy(a_hbm.at[di], ga, sem)
        cp_b = pltpu.async_copy(b_hbm.at[si], gb, sem)
        cp_a.wait()
        cp_b.wait()

        @pl.loop(0, K)
        def _row(r):
            for j in range(D // L):
                sl = pl.ds(j * L, L)
                ga[r, sl] = ga[r, sl] + gb[r, sl]

        pltpu.sync_copy(ga, out_hbm.at[pl.ds(off, K)])


@functools.partial(
    pl.kernel,
    out_type=jax.ShapeDtypeStruct((NC, N, D), jnp.float32),
    mesh=_mesh,
    scratch_types=[
        pltpu.VMEM((K,), jnp.int32),
        pltpu.VMEM((K, D), jnp.float32),
        pltpu.VMEM((ZR, D), jnp.float32),
        pltpu.VMEM_SHARED((N, D), jnp.float32),
        pltpu.SemaphoreType.DMA,
    ],
)
def _scatter_add(m2_hbm, dst_hbm, out_hbm, idxb, mb, zb, aggr, sem):
    cid = lax.axis_index("c")
    sid = lax.axis_index("s")
    wid = sid * NC + cid
    base = wid * EW

    @pl.loop(0, ZR)
    def _zrow(r):
        for j in range(D // L):
            zb[r, pl.ds(j * L, L)] = jnp.zeros((L,), jnp.float32)

    @pl.loop(0, RPT // ZR)
    def _zcopy(z):
        pltpu.sync_copy(zb, aggr.at[pl.ds(sid * RPT + z * ZR, ZR)])

    @pl.when(sid == NS - 1)
    def _ztail():
        pltpu.sync_copy(zb, aggr.at[pl.ds(NS * RPT, REM)])

    plsc.subcore_barrier()

    @pl.loop(0, NCHUNK)
    def _chunk(c):
        off = base + c * K
        pltpu.sync_copy(dst_hbm.at[pl.ds(off, K)], idxb)
        pltpu.sync_copy(m2_hbm.at[pl.ds(off, K)], mb)
        pltpu.sync_copy(mb, aggr.at[idxb], add=True)

    plsc.subcore_barrier()

    pltpu.sync_copy(aggr.at[pl.ds(sid * RPT, RPT)],
                    out_hbm.at[cid].at[pl.ds(sid * RPT, RPT)])

    @pl.when(sid == NS - 1)
    def _wtail():
        pltpu.sync_copy(aggr.at[pl.ds(NS * RPT, REM)],
                        out_hbm.at[cid].at[pl.ds(NS * RPT, REM)])


def _dot(x, w):
    return jnp.dot(x, w, precision=_PREC, preferred_element_type=jnp.float32)


def _pre_body(h_ref, wa_ref, wb_ref, a_ref, b_ref):
    hblk = h_ref[...]
    a_ref[...] = _dot(hblk, wa_ref[...])
    b_ref[...] = _dot(hblk, wb_ref[...])


def _precompute(h, w1a, w1b):
    blk = 1000
    grid = N // blk
    return pl.pallas_call(
        _pre_body,
        grid=(grid,),
        in_specs=[
            pl.BlockSpec((blk, D), lambda i: (i, 0)),
            pl.BlockSpec((D, D), lambda i: (0, 0)),
            pl.BlockSpec((D, D), lambda i: (0, 0)),
        ],
        out_specs=[
            pl.BlockSpec((blk, D), lambda i: (i, 0)),
            pl.BlockSpec((blk, D), lambda i: (i, 0)),
        ],
        out_shape=[
            jax.ShapeDtypeStruct((N, D), jnp.float32),
            jax.ShapeDtypeStruct((N, D), jnp.float32),
        ],
        compiler_params=pltpu.CompilerParams(
            dimension_semantics=("arbitrary",)),
    )(h, w1a, w1b)


def _msg_body(g_ref, ea_ref, w1c_ref, b1_ref, w2_ref, b2_ref, o_ref):
    x = g_ref[...] + _dot(ea_ref[...], w1c_ref[...]) + b1_ref[...]
    x = jnp.maximum(x, 0.0)
    y = _dot(x, w2_ref[...]) + b2_ref[...]
    o_ref[...] = jnp.maximum(y, 0.0)


def _msg(g, ea, w1c, b1, w2, b2):
    blk = 2000
    grid = E // blk
    return pl.pallas_call(
        _msg_body,
        grid=(grid,),
        in_specs=[
            pl.BlockSpec((blk, D), lambda i: (i, 0)),
            pl.BlockSpec((blk, DE), lambda i: (i, 0)),
            pl.BlockSpec((DE, D), lambda i: (0, 0)),
            pl.BlockSpec((1, D), lambda i: (0, 0)),
            pl.BlockSpec((D, D), lambda i: (0, 0)),
            pl.BlockSpec((1, D), lambda i: (0, 0)),
        ],
        out_specs=pl.BlockSpec((blk, D), lambda i: (i, 0)),
        out_shape=jax.ShapeDtypeStruct((E, D), jnp.float32),
        compiler_params=pltpu.CompilerParams(
            dimension_semantics=("arbitrary",)),
    )(g, ea, w1c, b1, w2, b2)


def _upd_body(h_ref, p0_ref, p1_ref, ua_ref, ub_ref, bu1_ref, u2_ref,
              bu2_ref, o_ref):
    aggr = p0_ref[...] + p1_ref[...]
    u = _dot(h_ref[...], ua_ref[...]) + _dot(aggr, ub_ref[...]) + bu1_ref[...]
    u = jnp.maximum(u, 0.0)
    o_ref[...] = jnp.maximum(_dot(u, u2_ref[...]) + bu2_ref[...], 0.0)


def _update(h, p0, p1, ua, ub, bu1, u2, bu2):
    blk = 1000
    grid = N // blk
    return pl.pallas_call(
        _upd_body,
        grid=(grid,),
        in_specs=[
            pl.BlockSpec((blk, D), lambda i: (i, 0)),
            pl.BlockSpec((blk, D), lambda i: (i, 0)),
            pl.BlockSpec((blk, D), lambda i: (i, 0)),
            pl.BlockSpec((D, D), lambda i: (0, 0)),
            pl.BlockSpec((D, D), lambda i: (0, 0)),
            pl.BlockSpec((1, D), lambda i: (0, 0)),
            pl.BlockSpec((D, D), lambda i: (0, 0)),
            pl.BlockSpec((1, D), lambda i: (0, 0)),
        ],
        out_specs=pl.BlockSpec((blk, D), lambda i: (i, 0)),
        out_shape=jax.ShapeDtypeStruct((N, D), jnp.float32),
        compiler_params=pltpu.CompilerParams(
            dimension_semantics=("arbitrary",)),
    )(h, p0, p1, ua, ub, bu1, u2, bu2)


def kernel(h, edge_index, edge_attr, W1, b1, W2, b2, U1, bu1, U2, bu2):
    src = edge_index[0]
    dst = edge_index[1]
    w1a, w1b, w1c = W1[:D], W1[D:2 * D], W1[2 * D:]
    ua, ub = U1[:D], U1[D:]
    b1r = b1.reshape(1, D)
    b2r = b2.reshape(1, D)
    bu1r = bu1.reshape(1, D)
    bu2r = bu2.reshape(1, D)

    a, b = _precompute(h, w1a, w1b)
    g = _gather_sum(a, b, dst, src)
    m2 = _msg(g, edge_attr, w1c, b1r, W2, b2r)
    parts = _scatter_add(m2, dst)
    out = _update(h, parts[0], parts[1], ua, ub, bu1r, U2, bu2r)
    return out


# speedup vs baseline: 2.5131x; 1.1329x over previous
import functools

import jax
import jax.numpy as jnp
from jax import lax
from jax.experimental import pallas as pl
from jax.experimental.pallas import tpu as pltpu
from jax.experimental.pallas import tpu_sc as plsc

N, E, D, DE = 10000, 320000, 128, 16
NC, NS, L = 2, 16, 16
NW = NC * NS
EW = E // NW
K = 80
NCHUNK = EW // K
RPT = 624
REM = N - NS * RPT
ZR = 16

_PREC = lax.Precision.HIGHEST

_mesh = plsc.VectorSubcoreMesh(
    core_axis_name="c", subcore_axis_name="s", num_cores=NC, num_subcores=NS)


@functools.partial(
    pl.kernel,
    out_type=jax.ShapeDtypeStruct((E, D), jnp.float32),
    mesh=_mesh,
    scratch_types=[
        pltpu.VMEM((EW,), jnp.int32),
        pltpu.VMEM((EW,), jnp.int32),
        pltpu.VMEM((K, D), jnp.float32),
        pltpu.VMEM((K, D), jnp.float32),
        pltpu.VMEM((K, D), jnp.float32),
        pltpu.VMEM((K, D), jnp.float32),
        pltpu.SemaphoreType.DMA,
        pltpu.SemaphoreType.DMA,
        pltpu.SemaphoreType.DMA,
        pltpu.SemaphoreType.DMA,
    ],
)
def _gather_sum(a_hbm, b_hbm, dst_hbm, src_hbm, out_hbm,
                dstv, srcv, ga0, gb0, ga1, gb1, sg0, sg1, so0, so1):
    wid = lax.axis_index("s") * NC + lax.axis_index("c")
    base = wid * EW

    pltpu.sync_copy(dst_hbm.at[pl.ds(base, EW)], dstv)
    pltpu.sync_copy(src_hbm.at[pl.ds(base, EW)], srcv)

    bufs = ((ga0, gb0, sg0, so0), (ga1, gb1, sg1, so1))

    def gstart(c, b):
        ga, gb, sg, _ = bufs[b]
        pltpu.async_copy(a_hbm.at[dstv.at[pl.ds(c * K, K)]], ga, sg)
        pltpu.async_copy(b_hbm.at[srcv.at[pl.ds(c * K, K)]], gb, sg)

    def gwait(b):
        ga, gb, sg, _ = bufs[b]
        pltpu.make_async_copy(a_hbm.at[dstv.at[pl.ds(0, K)]], ga, sg).wait()
        pltpu.make_async_copy(b_hbm.at[srcv.at[pl.ds(0, K)]], gb, sg).wait()

    def add(b):
        ga, gb, _, _ = bufs[b]

        @pl.loop(0, K, unroll=4)
        def _row(r):
            for j in range(D // L):
                sl = pl.ds(j * L, L)
                ga[r, sl] = ga[r, sl] + gb[r, sl]

    def ostart(c, b):
        ga, _, _, so = bufs[b]
        pltpu.async_copy(ga, out_hbm.at[pl.ds(base + c * K, K)], so)

    def owait(b):
        ga, _, _, so = bufs[b]
        pltpu.make_async_copy(ga, out_hbm.at[pl.ds(base, K)], so).wait()

    gstart(0, 0)
    gwait(0)
    gstart(1, 1)
    add(0)
    ostart(0, 0)

    @pl.loop(0, (NCHUNK - 3) // 2)
    def _pair(p):
        c1 = 2 * p + 1
        gwait(1)
        owait(0)
        gstart(c1 + 1, 0)
        add(1)
        ostart(c1, 1)
        c2 = 2 * p + 2
        gwait(0)
        owait(1)
        gstart(c2 + 1, 1)
        add(0)
        ostart(c2, 0)

    gwait(1)
    owait(0)
    gstart(NCHUNK - 1, 0)
    add(1)
    ostart(NCHUNK - 2, 1)
    gwait(0)
    owait(1)
    add(0)
    ostart(NCHUNK - 1, 0)
    owait(0)


@functools.partial(
    pl.kernel,
    out_type=jax.ShapeDtypeStruct((NC, N, D), jnp.float32),
    mesh=_mesh,
    scratch_types=[
        pltpu.VMEM((K,), jnp.int32),
        pltpu.VMEM((K,), jnp.int32),
        pltpu.VMEM((K, D), jnp.float32),
        pltpu.VMEM((K, D), jnp.float32),
        pltpu.VMEM((ZR, D), jnp.float32),
        pltpu.VMEM_SHARED((N, D), jnp.float32),
        pltpu.SemaphoreType.DMA,
        pltpu.SemaphoreType.DMA,
        pltpu.SemaphoreType.DMA,
        pltpu.SemaphoreType.DMA,
    ],
)
def _scatter_add(m2_hbm, dst_hbm, out_hbm,
                 idx0, idx1, mb0, mb1, zb, aggr, sl0, sl1, ss0, ss1):
    cid = lax.axis_index("c")
    sid = lax.axis_index("s")
    wid = sid * NC + cid
    base = wid * EW

    @pl.loop(0, ZR)
    def _zrow(r):
        for j in range(D // L):
            zb[r, pl.ds(j * L, L)] = jnp.zeros((L,), jnp.float32)

    @pl.loop(0, RPT // ZR)
    def _zcopy(z):
        pltpu.sync_copy(zb, aggr.at[pl.ds(sid * RPT + z * ZR, ZR)])

    @pl.when(sid == NS - 1)
    def _ztail():
        pltpu.sync_copy(zb, aggr.at[pl.ds(NS * RPT, REM)])

    plsc.subcore_barrier()

    bufs = ((idx0, mb0, sl0, ss0), (idx1, mb1, sl1, ss1))

    def lstart(c, b):
        idxb, mb, sl, _ = bufs[b]
        off = base + c * K
        pltpu.async_copy(dst_hbm.at[pl.ds(off, K)], idxb, sl)
        pltpu.async_copy(m2_hbm.at[pl.ds(off, K)], mb, sl)

    def lwait(b):
        idxb, mb, sl, _ = bufs[b]
        pltpu.make_async_copy(dst_hbm.at[pl.ds(base, K)], idxb, sl).wait()
        pltpu.make_async_copy(m2_hbm.at[pl.ds(base, K)], mb, sl).wait()

    def sstart(b):
        idxb, mb, _, ss = bufs[b]
        pltpu.async_copy(mb, aggr.at[idxb], ss, add=True)

    def swait(b):
        idxb, mb, _, ss = bufs[b]
        pltpu.make_async_copy(mb, aggr.at[idxb], ss).wait()

    lstart(0, 0)
    lwait(0)
    lstart(1, 1)
    sstart(0)

    @pl.loop(0, (NCHUNK - 3) // 2)
    def _pair(p):
        lwait(1)
        swait(0)
        lstart(2 * p + 2, 0)
        sstart(1)
        lwait(0)
        swait(1)
        lstart(2 * p + 3, 1)
        sstart(0)

    lwait(1)
    swait(0)
    lstart(NCHUNK - 1, 0)
    sstart(1)
    lwait(0)
    swait(1)
    sstart(0)
    swait(0)

    plsc.subcore_barrier()

    pltpu.sync_copy(aggr.at[pl.ds(sid * RPT, RPT)],
                    out_hbm.at[cid].at[pl.ds(sid * RPT, RPT)])

    @pl.when(sid == NS - 1)
    def _wtail():
        pltpu.sync_copy(aggr.at[pl.ds(NS * RPT, REM)],
                        out_hbm.at[cid].at[pl.ds(NS * RPT, REM)])


def _dot(x, w):
    return jnp.dot(x, w, precision=_PREC, preferred_element_type=jnp.float32)


def _pre_body(h_ref, wa_ref, wb_ref, a_ref, b_ref):
    hblk = h_ref[...]
    a_ref[...] = _dot(hblk, wa_ref[...])
    b_ref[...] = _dot(hblk, wb_ref[...])


def _precompute(h, w1a, w1b):
    blk = 1000
    grid = N // blk
    return pl.pallas_call(
        _pre_body,
        grid=(grid,),
        in_specs=[
            pl.BlockSpec((blk, D), lambda i: (i, 0)),
            pl.BlockSpec((D, D), lambda i: (0, 0)),
            pl.BlockSpec((D, D), lambda i: (0, 0)),
        ],
        out_specs=[
            pl.BlockSpec((blk, D), lambda i: (i, 0)),
            pl.BlockSpec((blk, D), lambda i: (i, 0)),
        ],
        out_shape=[
            jax.ShapeDtypeStruct((N, D), jnp.float32),
            jax.ShapeDtypeStruct((N, D), jnp.float32),
        ],
        compiler_params=pltpu.CompilerParams(
            dimension_semantics=("arbitrary",)),
    )(h, w1a, w1b)


def _msg_body(g_ref, ea_ref, w1c_ref, b1_ref, w2_ref, b2_ref, o_ref):
    x = g_ref[...] + _dot(ea_ref[...], w1c_ref[...]) + b1_ref[...]
    x = jnp.maximum(x, 0.0)
    y = _dot(x, w2_ref[...]) + b2_ref[...]
    o_ref[...] = jnp.maximum(y, 0.0)


def _msg(g, ea, w1c, b1, w2, b2):
    blk = 2000
    grid = E // blk
    return pl.pallas_call(
        _msg_body,
        grid=(grid,),
        in_specs=[
            pl.BlockSpec((blk, D), lambda i: (i, 0)),
            pl.BlockSpec((blk, DE), lambda i: (i, 0)),
            pl.BlockSpec((DE, D), lambda i: (0, 0)),
            pl.BlockSpec((1, D), lambda i: (0, 0)),
            pl.BlockSpec((D, D), lambda i: (0, 0)),
            pl.BlockSpec((1, D), lambda i: (0, 0)),
        ],
        out_specs=pl.BlockSpec((blk, D), lambda i: (i, 0)),
        out_shape=jax.ShapeDtypeStruct((E, D), jnp.float32),
        compiler_params=pltpu.CompilerParams(
            dimension_semantics=("arbitrary",)),
    )(g, ea, w1c, b1, w2, b2)


def _upd_body(h_ref, p0_ref, p1_ref, ua_ref, ub_ref, bu1_ref, u2_ref,
              bu2_ref, o_ref):
    aggr = p0_ref[...] + p1_ref[...]
    u = _dot(h_ref[...], ua_ref[...]) + _dot(aggr, ub_ref[...]) + bu1_ref[...]
    u = jnp.maximum(u, 0.0)
    o_ref[...] = jnp.maximum(_dot(u, u2_ref[...]) + bu2_ref[...], 0.0)


def _update(h, p0, p1, ua, ub, bu1, u2, bu2):
    blk = 1000
    grid = N // blk
    return pl.pallas_call(
        _upd_body,
        grid=(grid,),
        in_specs=[
            pl.BlockSpec((blk, D), lambda i: (i, 0)),
            pl.BlockSpec((blk, D), lambda i: (i, 0)),
            pl.BlockSpec((blk, D), lambda i: (i, 0)),
            pl.BlockSpec((D, D), lambda i: (0, 0)),
            pl.BlockSpec((D, D), lambda i: (0, 0)),
            pl.BlockSpec((1, D), lambda i: (0, 0)),
            pl.BlockSpec((D, D), lambda i: (0, 0)),
            pl.BlockSpec((1, D), lambda i: (0, 0)),
        ],
        out_specs=pl.BlockSpec((blk, D), lambda i: (i, 0)),
        out_shape=jax.ShapeDtypeStruct((N, D), jnp.float32),
        compiler_params=pltpu.CompilerParams(
            dimension_semantics=("arbitrary",)),
    )(h, p0, p1, ua, ub, bu1, u2, bu2)


def kernel(h, edge_index, edge_attr, W1, b1, W2, b2, U1, bu1, U2, bu2):
    src = edge_index[0]
    dst = edge_index[1]
    w1a, w1b, w1c = W1[:D], W1[D:2 * D], W1[2 * D:]
    ua, ub = U1[:D], U1[D:]
    b1r = b1.reshape(1, D)
    b2r = b2.reshape(1, D)
    bu1r = bu1.reshape(1, D)
    bu2r = bu2.reshape(1, D)

    a, b = _precompute(h, w1a, w1b)
    g = _gather_sum(a, b, dst, src)
    m2 = _msg(g, edge_attr, w1c, b1r, W2, b2r)
    parts = _scatter_add(m2, dst)
    out = _update(h, parts[0], parts[1], ua, ub, bu1r, U2, bu2r)
    return out


# speedup vs baseline: 3.5095x; 1.3964x over previous
import functools

import jax
import jax.numpy as jnp
from jax import lax
from jax.experimental import pallas as pl
from jax.experimental.pallas import tpu as pltpu
from jax.experimental.pallas import tpu_sc as plsc

N, E, D, DE = 10000, 320000, 128, 16
NC, NS, L = 2, 16, 16
NW = NC * NS
EW = E // NW
K = 80
NCHUNK = EW // K
RPT = 624
REM = N - NS * RPT
ZR = 16

_PREC = lax.Precision.DEFAULT

_mesh = plsc.VectorSubcoreMesh(
    core_axis_name="c", subcore_axis_name="s", num_cores=NC, num_subcores=NS)


@functools.partial(
    pl.kernel,
    out_type=jax.ShapeDtypeStruct((E, D), jnp.float32),
    mesh=_mesh,
    scratch_types=[
        pltpu.VMEM((EW,), jnp.int32),
        pltpu.VMEM((EW,), jnp.int32),
        pltpu.VMEM((K, D), jnp.float32),
        pltpu.VMEM((K, D), jnp.float32),
        pltpu.VMEM((K, D), jnp.float32),
        pltpu.VMEM((K, D), jnp.float32),
        pltpu.SemaphoreType.DMA,
        pltpu.SemaphoreType.DMA,
        pltpu.SemaphoreType.DMA,
        pltpu.SemaphoreType.DMA,
    ],
)
def _gather_sum(a_hbm, b_hbm, dst_hbm, src_hbm, out_hbm,
                dstv, srcv, ga0, gb0, ga1, gb1, sg0, sg1, so0, so1):
    wid = lax.axis_index("s") * NC + lax.axis_index("c")
    base = wid * EW

    pltpu.sync_copy(dst_hbm.at[pl.ds(base, EW)], dstv)
    pltpu.sync_copy(src_hbm.at[pl.ds(base, EW)], srcv)

    bufs = ((ga0, gb0, sg0, so0), (ga1, gb1, sg1, so1))

    def gstart(c, b):
        ga, gb, sg, _ = bufs[b]
        pltpu.async_copy(a_hbm.at[dstv.at[pl.ds(c * K, K)]], ga, sg)
        pltpu.async_copy(b_hbm.at[srcv.at[pl.ds(c * K, K)]], gb, sg)

    def gwait(b):
        ga, gb, sg, _ = bufs[b]
        pltpu.make_async_copy(a_hbm.at[dstv.at[pl.ds(0, K)]], ga, sg).wait()
        pltpu.make_async_copy(b_hbm.at[srcv.at[pl.ds(0, K)]], gb, sg).wait()

    def add(b):
        ga, gb, _, _ = bufs[b]

        @pl.loop(0, K, unroll=4)
        def _row(r):
            for j in range(D // L):
                sl = pl.ds(j * L, L)
                ga[r, sl] = ga[r, sl] + gb[r, sl]

    def ostart(c, b):
        ga, _, _, so = bufs[b]
        pltpu.async_copy(ga, out_hbm.at[pl.ds(base + c * K, K)], so)

    def owait(b):
        ga, _, _, so = bufs[b]
        pltpu.make_async_copy(ga, out_hbm.at[pl.ds(base, K)], so).wait()

    gstart(0, 0)
    gwait(0)
    gstart(1, 1)
    add(0)
    ostart(0, 0)

    @pl.loop(0, (NCHUNK - 3) // 2)
    def _pair(p):
        c1 = 2 * p + 1
        gwait(1)
        owait(0)
        gstart(c1 + 1, 0)
        add(1)
        ostart(c1, 1)
        c2 = 2 * p + 2
        gwait(0)
        owait(1)
        gstart(c2 + 1, 1)
        add(0)
        ostart(c2, 0)

    gwait(1)
    owait(0)
    gstart(NCHUNK - 1, 0)
    add(1)
    ostart(NCHUNK - 2, 1)
    gwait(0)
    owait(1)
    add(0)
    ostart(NCHUNK - 1, 0)
    owait(0)


@functools.partial(
    pl.kernel,
    out_type=jax.ShapeDtypeStruct((NC, N, D), jnp.float32),
    mesh=_mesh,
    scratch_types=[
        pltpu.VMEM((K,), jnp.int32),
        pltpu.VMEM((K,), jnp.int32),
        pltpu.VMEM((K, D), jnp.float32),
        pltpu.VMEM((K, D), jnp.float32),
        pltpu.VMEM((ZR, D), jnp.float32),
        pltpu.VMEM_SHARED((N, D), jnp.float32),
        pltpu.SemaphoreType.DMA,
        pltpu.SemaphoreType.DMA,
        pltpu.SemaphoreType.DMA,
        pltpu.SemaphoreType.DMA,
    ],
)
def _scatter_add(m2_hbm, dst_hbm, out_hbm,
                 idx0, idx1, mb0, mb1, zb, aggr, sl0, sl1, ss0, ss1):
    cid = lax.axis_index("c")
    sid = lax.axis_index("s")
    wid = sid * NC + cid
    base = wid * EW

    @pl.loop(0, ZR)
    def _zrow(r):
        for j in range(D // L):
            zb[r, pl.ds(j * L, L)] = jnp.zeros((L,), jnp.float32)

    @pl.loop(0, RPT // ZR)
    def _zcopy(z):
        pltpu.sync_copy(zb, aggr.at[pl.ds(sid * RPT + z * ZR, ZR)])

    @pl.when(sid == NS - 1)
    def _ztail():
        pltpu.sync_copy(zb, aggr.at[pl.ds(NS * RPT, REM)])

    plsc.subcore_barrier()

    bufs = ((idx0, mb0, sl0, ss0), (idx1, mb1, sl1, ss1))

    def lstart(c, b):
        idxb, mb, sl, _ = bufs[b]
        off = base + c * K
        pltpu.async_copy(dst_hbm.at[pl.ds(off, K)], idxb, sl)
        pltpu.async_copy(m2_hbm.at[pl.ds(off, K)], mb, sl)

    def lwait(b):
        idxb, mb, sl, _ = bufs[b]
        pltpu.make_async_copy(dst_hbm.at[pl.ds(base, K)], idxb, sl).wait()
        pltpu.make_async_copy(m2_hbm.at[pl.ds(base, K)], mb, sl).wait()

    def sstart(b):
        idxb, mb, _, ss = bufs[b]
        pltpu.async_copy(mb, aggr.at[idxb], ss, add=True)

    def swait(b):
        idxb, mb, _, ss = bufs[b]
        pltpu.make_async_copy(mb, aggr.at[idxb], ss).wait()

    lstart(0, 0)
    lwait(0)
    lstart(1, 1)
    sstart(0)

    @pl.loop(0, (NCHUNK - 3) // 2)
    def _pair(p):
        lwait(1)
        swait(0)
        lstart(2 * p + 2, 0)
        sstart(1)
        lwait(0)
        swait(1)
        lstart(2 * p + 3, 1)
        sstart(0)

    lwait(1)
    swait(0)
    lstart(NCHUNK - 1, 0)
    sstart(1)
    lwait(0)
    swait(1)
    sstart(0)
    swait(0)

    plsc.subcore_barrier()

    pltpu.sync_copy(aggr.at[pl.ds(sid * RPT, RPT)],
                    out_hbm.at[cid].at[pl.ds(sid * RPT, RPT)])

    @pl.when(sid == NS - 1)
    def _wtail():
        pltpu.sync_copy(aggr.at[pl.ds(NS * RPT, REM)],
                        out_hbm.at[cid].at[pl.ds(NS * RPT, REM)])


def _dot(x, w):
    return jnp.dot(x, w, precision=_PREC, preferred_element_type=jnp.float32)


def _pre_body(h_ref, wa_ref, wb_ref, a_ref, b_ref):
    hblk = h_ref[...]
    a_ref[...] = _dot(hblk, wa_ref[...])
    b_ref[...] = _dot(hblk, wb_ref[...])


def _precompute(h, w1a, w1b):
    blk = 1000
    grid = N // blk
    return pl.pallas_call(
        _pre_body,
        grid=(grid,),
        in_specs=[
            pl.BlockSpec((blk, D), lambda i: (i, 0)),
            pl.BlockSpec((D, D), lambda i: (0, 0)),
            pl.BlockSpec((D, D), lambda i: (0, 0)),
        ],
        out_specs=[
            pl.BlockSpec((blk, D), lambda i: (i, 0)),
            pl.BlockSpec((blk, D), lambda i: (i, 0)),
        ],
        out_shape=[
            jax.ShapeDtypeStruct((N, D), jnp.float32),
            jax.ShapeDtypeStruct((N, D), jnp.float32),
        ],
        compiler_params=pltpu.CompilerParams(
            dimension_semantics=("arbitrary",)),
    )(h, w1a, w1b)


def _msg_body(g_ref, ea_ref, w1c_ref, b1_ref, w2_ref, b2_ref, o_ref):
    x = g_ref[...] + _dot(ea_ref[...], w1c_ref[...]) + b1_ref[...]
    x = jnp.maximum(x, 0.0)
    y = _dot(x, w2_ref[...]) + b2_ref[...]
    o_ref[...] = jnp.maximum(y, 0.0)


def _msg(g, ea, w1c, b1, w2, b2):
    blk = 2000
    grid = E // blk
    return pl.pallas_call(
        _msg_body,
        grid=(grid,),
        in_specs=[
            pl.BlockSpec((blk, D), lambda i: (i, 0)),
            pl.BlockSpec((blk, DE), lambda i: (i, 0)),
            pl.BlockSpec((DE, D), lambda i: (0, 0)),
            pl.BlockSpec((1, D), lambda i: (0, 0)),
            pl.BlockSpec((D, D), lambda i: (0, 0)),
            pl.BlockSpec((1, D), lambda i: (0, 0)),
        ],
        out_specs=pl.BlockSpec((blk, D), lambda i: (i, 0)),
        out_shape=jax.ShapeDtypeStruct((E, D), jnp.float32),
        compiler_params=pltpu.CompilerParams(
            dimension_semantics=("arbitrary",)),
    )(g, ea, w1c, b1, w2, b2)


def _upd_body(h_ref, p0_ref, p1_ref, ua_ref, ub_ref, bu1_ref, u2_ref,
              bu2_ref, o_ref):
    aggr = p0_ref[...] + p1_ref[...]
    u = _dot(h_ref[...], ua_ref[...]) + _dot(aggr, ub_ref[...]) + bu1_ref[...]
    u = jnp.maximum(u, 0.0)
    o_ref[...] = jnp.maximum(_dot(u, u2_ref[...]) + bu2_ref[...], 0.0)


def _update(h, p0, p1, ua, ub, bu1, u2, bu2):
    blk = 1000
    grid = N // blk
    return pl.pallas_call(
        _upd_body,
        grid=(grid,),
        in_specs=[
            pl.BlockSpec((blk, D), lambda i: (i, 0)),
            pl.BlockSpec((blk, D), lambda i: (i, 0)),
            pl.BlockSpec((blk, D), lambda i: (i, 0)),
            pl.BlockSpec((D, D), lambda i: (0, 0)),
            pl.BlockSpec((D, D), lambda i: (0, 0)),
            pl.BlockSpec((1, D), lambda i: (0, 0)),
            pl.BlockSpec((D, D), lambda i: (0, 0)),
            pl.BlockSpec((1, D), lambda i: (0, 0)),
        ],
        out_specs=pl.BlockSpec((blk, D), lambda i: (i, 0)),
        out_shape=jax.ShapeDtypeStruct((N, D), jnp.float32),
        compiler_params=pltpu.CompilerParams(
            dimension_semantics=("arbitrary",)),
    )(h, p0, p1, ua, ub, bu1, u2, bu2)


def kernel(h, edge_index, edge_attr, W1, b1, W2, b2, U1, bu1, U2, bu2):
    src = edge_index[0]
    dst = edge_index[1]
    w1a, w1b, w1c = W1[:D], W1[D:2 * D], W1[2 * D:]
    ua, ub = U1[:D], U1[D:]
    b1r = b1.reshape(1, D)
    b2r = b2.reshape(1, D)
    bu1r = bu1.reshape(1, D)
    bu2r = bu2.reshape(1, D)

    a, b = _precompute(h, w1a, w1b)
    g = _gather_sum(a, b, dst, src)
    m2 = _msg(g, edge_attr, w1c, b1r, W2, b2r)
    parts = _scatter_add(m2, dst)
    out = _update(h, parts[0], parts[1], ua, ub, bu1r, U2, bu2r)
    return out


# speedup vs baseline: 3.7304x; 1.0630x over previous
import functools

import jax
import jax.numpy as jnp
from jax import lax
from jax.experimental import pallas as pl
from jax.experimental.pallas import tpu as pltpu
from jax.experimental.pallas import tpu_sc as plsc

N, E, D, DE = 10000, 320000, 128, 16
NC, NS, L = 2, 16, 16
NW = NC * NS
RPT = 624
REM = N - NS * RPT
ZR = 16
NSPLIT = 2

_PREC = lax.Precision.DEFAULT

_mesh = plsc.VectorSubcoreMesh(
    core_axis_name="c", subcore_axis_name="s", num_cores=NC, num_subcores=NS)


def _pick_k(ew):
    for k in range(128, 7, -8):
        if ew % k == 0 and (ew // k) % 2 == 1 and ew // k >= 5:
            return k
    raise ValueError(f"no chunk size for per-worker range {ew}")


@functools.cache
def _make_gather(ecnt):
    ew = ecnt // NW
    k = _pick_k(ew)
    nchunk = ew // k

    @functools.partial(
        pl.kernel,
        out_type=jax.ShapeDtypeStruct((ecnt, D), jnp.float32),
        mesh=_mesh,
        scratch_types=[
            pltpu.VMEM((ew,), jnp.int32),
            pltpu.VMEM((ew,), jnp.int32),
            pltpu.VMEM((k, D), jnp.float32),
            pltpu.VMEM((k, D), jnp.float32),
            pltpu.VMEM((k, D), jnp.float32),
            pltpu.VMEM((k, D), jnp.float32),
            pltpu.SemaphoreType.DMA,
            pltpu.SemaphoreType.DMA,
            pltpu.SemaphoreType.DMA,
            pltpu.SemaphoreType.DMA,
        ],
    )
    def _gather_sum(a_hbm, b_hbm, dst_hbm, src_hbm, out_hbm,
                    dstv, srcv, ga0, gb0, ga1, gb1, sg0, sg1, so0, so1):
        wid = lax.axis_index("s") * NC + lax.axis_index("c")
        base = wid * ew

        pltpu.sync_copy(dst_hbm.at[pl.ds(base, ew)], dstv)
        pltpu.sync_copy(src_hbm.at[pl.ds(base, ew)], srcv)

        bufs = ((ga0, gb0, sg0, so0), (ga1, gb1, sg1, so1))

        def gstart(c, b):
            ga, gb, sg, _ = bufs[b]
            pltpu.async_copy(a_hbm.at[dstv.at[pl.ds(c * k, k)]], ga, sg)
            pltpu.async_copy(b_hbm.at[srcv.at[pl.ds(c * k, k)]], gb, sg)

        def gwait(b):
            ga, gb, sg, _ = bufs[b]
            pltpu.make_async_copy(
                a_hbm.at[dstv.at[pl.ds(0, k)]], ga, sg).wait()
            pltpu.make_async_copy(
                b_hbm.at[srcv.at[pl.ds(0, k)]], gb, sg).wait()

        def add(b):
            ga, gb, _, _ = bufs[b]

            @pl.loop(0, k, unroll=4)
            def _row(r):
                for j in range(D // L):
                    sl = pl.ds(j * L, L)
                    ga[r, sl] = ga[r, sl] + gb[r, sl]

        def ostart(c, b):
            ga, _, _, so = bufs[b]
            pltpu.async_copy(ga, out_hbm.at[pl.ds(base + c * k, k)], so)

        def owait(b):
            ga, _, _, so = bufs[b]
            pltpu.make_async_copy(ga, out_hbm.at[pl.ds(base, k)], so).wait()

        gstart(0, 0)
        gwait(0)
        gstart(1, 1)
        add(0)
        ostart(0, 0)

        @pl.loop(0, (nchunk - 3) // 2)
        def _pair(p):
            c1 = 2 * p + 1
            gwait(1)
            owait(0)
            gstart(c1 + 1, 0)
            add(1)
            ostart(c1, 1)
            c2 = 2 * p + 2
            gwait(0)
            owait(1)
            gstart(c2 + 1, 1)
            add(0)
            ostart(c2, 0)

        gwait(1)
        owait(0)
        gstart(nchunk - 1, 0)
        add(1)
        ostart(nchunk - 2, 1)
        gwait(0)
        owait(1)
        add(0)
        ostart(nchunk - 1, 0)
        owait(0)

    return _gather_sum


@functools.cache
def _make_scatter(ecnt):
    ew = ecnt // NW
    k = _pick_k(ew)
    nchunk = ew // k

    @functools.partial(
        pl.kernel,
        out_type=jax.ShapeDtypeStruct((NC, N, D), jnp.float32),
        mesh=_mesh,
        scratch_types=[
            pltpu.VMEM((k,), jnp.int32),
            pltpu.VMEM((k,), jnp.int32),
            pltpu.VMEM((k, D), jnp.float32),
            pltpu.VMEM((k, D), jnp.float32),
            pltpu.VMEM((ZR, D), jnp.float32),
            pltpu.VMEM_SHARED((N, D), jnp.float32),
            pltpu.SemaphoreType.DMA,
            pltpu.SemaphoreType.DMA,
            pltpu.SemaphoreType.DMA,
            pltpu.SemaphoreType.DMA,
        ],
    )
    def _scatter_add(m2_hbm, dst_hbm, out_hbm,
                     idx0, idx1, mb0, mb1, zb, aggr, sl0, sl1, ss0, ss1):
        cid = lax.axis_index("c")
        sid = lax.axis_index("s")
        wid = sid * NC + cid
        base = wid * ew

        @pl.loop(0, ZR)
        def _zrow(r):
            for j in range(D // L):
                zb[r, pl.ds(j * L, L)] = jnp.zeros((L,), jnp.float32)

        @pl.loop(0, RPT // ZR)
        def _zcopy(z):
            pltpu.sync_copy(zb, aggr.at[pl.ds(sid * RPT + z * ZR, ZR)])

        @pl.when(sid == NS - 1)
        def _ztail():
            pltpu.sync_copy(zb, aggr.at[pl.ds(NS * RPT, REM)])

        plsc.subcore_barrier()

        bufs = ((idx0, mb0, sl0, ss0), (idx1, mb1, sl1, ss1))

        def lstart(c, b):
            idxb, mb, sl, _ = bufs[b]
            off = base + c * k
            pltpu.async_copy(dst_hbm.at[pl.ds(off, k)], idxb, sl)
            pltpu.async_copy(m2_hbm.at[pl.ds(off, k)], mb, sl)

        def lwait(b):
            idxb, mb, sl, _ = bufs[b]
            pltpu.make_async_copy(dst_hbm.at[pl.ds(base, k)], idxb, sl).wait()
            pltpu.make_async_copy(m2_hbm.at[pl.ds(base, k)], mb, sl).wait()

        def sstart(b):
            idxb, mb, _, ss = bufs[b]
            pltpu.async_copy(mb, aggr.at[idxb], ss, add=True)

        def swait(b):
            idxb, mb, _, ss = bufs[b]
            pltpu.make_async_copy(mb, aggr.at[idxb], ss).wait()

        lstart(0, 0)
        lwait(0)
        lstart(1, 1)
        sstart(0)

        @pl.loop(0, (nchunk - 3) // 2)
        def _pair(p):
            lwait(1)
            swait(0)
            lstart(2 * p + 2, 0)
            sstart(1)
            lwait(0)
            swait(1)
            lstart(2 * p + 3, 1)
            sstart(0)

        lwait(1)
        swait(0)
        lstart(nchunk - 1, 0)
        sstart(1)
        lwait(0)
        swait(1)
        sstart(0)
        swait(0)

        plsc.subcore_barrier()

        pltpu.sync_copy(aggr.at[pl.ds(sid * RPT, RPT)],
                        out_hbm.at[cid].at[pl.ds(sid * RPT, RPT)])

        @pl.when(sid == NS - 1)
        def _wtail():
            pltpu.sync_copy(aggr.at[pl.ds(NS * RPT, REM)],
                            out_hbm.at[cid].at[pl.ds(NS * RPT, REM)])

    return _scatter_add


def _dot(x, w):
    return jnp.dot(x, w, precision=_PREC, preferred_element_type=jnp.float32)


def _pre_body(h_ref, wa_ref, wb_ref, a_ref, b_ref):
    hblk = h_ref[...]
    a_ref[...] = _dot(hblk, wa_ref[...])
    b_ref[...] = _dot(hblk, wb_ref[...])


def _precompute(h, w1a, w1b):
    blk = 1000
    grid = N // blk
    return pl.pallas_call(
        _pre_body,
        grid=(grid,),
        in_specs=[
            pl.BlockSpec((blk, D), lambda i: (i, 0)),
            pl.BlockSpec((D, D), lambda i: (0, 0)),
            pl.BlockSpec((D, D), lambda i: (0, 0)),
        ],
        out_specs=[
            pl.BlockSpec((blk, D), lambda i: (i, 0)),
            pl.BlockSpec((blk, D), lambda i: (i, 0)),
        ],
        out_shape=[
            jax.ShapeDtypeStruct((N, D), jnp.float32),
            jax.ShapeDtypeStruct((N, D), jnp.float32),
        ],
        compiler_params=pltpu.CompilerParams(
            dimension_semantics=("arbitrary",)),
    )(h, w1a, w1b)


def _msg_body(g_ref, ea_ref, w1c_ref, b1_ref, w2_ref, b2_ref, o_ref):
    x = g_ref[...] + _dot(ea_ref[...], w1c_ref[...]) + b1_ref[...]
    x = jnp.maximum(x, 0.0)
    y = _dot(x, w2_ref[...]) + b2_ref[...]
    o_ref[...] = jnp.maximum(y, 0.0)


def _msg(g, ea, w1c, b1, w2, b2):
    ecnt = g.shape[0]
    blk = 2000
    grid = ecnt // blk
    return pl.pallas_call(
        _msg_body,
        grid=(grid,),
        in_specs=[
            pl.BlockSpec((blk, D), lambda i: (i, 0)),
            pl.BlockSpec((blk, DE), lambda i: (i, 0)),
            pl.BlockSpec((DE, D), lambda i: (0, 0)),
            pl.BlockSpec((1, D), lambda i: (0, 0)),
            pl.BlockSpec((D, D), lambda i: (0, 0)),
            pl.BlockSpec((1, D), lambda i: (0, 0)),
        ],
        out_specs=pl.BlockSpec((blk, D), lambda i: (i, 0)),
        out_shape=jax.ShapeDtypeStruct((ecnt, D), jnp.float32),
        compiler_params=pltpu.CompilerParams(
            dimension_semantics=("arbitrary",)),
    )(g, ea, w1c, b1, w2, b2)


def _upd_body(*refs):
    h_ref = refs[0]
    parts = refs[1:1 + NSPLIT * NC]
    ua_ref, ub_ref, bu1_ref, u2_ref, bu2_ref, o_ref = refs[1 + NSPLIT * NC:]
    aggr = parts[0][...]
    for p_ref in parts[1:]:
        aggr = aggr + p_ref[...]
    u = _dot(h_ref[...], ua_ref[...]) + _dot(aggr, ub_ref[...]) + bu1_ref[...]
    u = jnp.maximum(u, 0.0)
    o_ref[...] = jnp.maximum(_dot(u, u2_ref[...]) + bu2_ref[...], 0.0)


def _update(h, parts, ua, ub, bu1, u2, bu2):
    blk = 1000
    grid = N // blk
    nblk = pl.BlockSpec((blk, D), lambda i: (i, 0))
    wblk = pl.BlockSpec((D, D), lambda i: (0, 0))
    bblk = pl.BlockSpec((1, D), lambda i: (0, 0))
    return pl.pallas_call(
        _upd_body,
        grid=(grid,),
        in_specs=[nblk] + [nblk] * len(parts) + [wblk, wblk, bblk, wblk, bblk],
        out_specs=nblk,
        out_shape=jax.ShapeDtypeStruct((N, D), jnp.float32),
        compiler_params=pltpu.CompilerParams(
            dimension_semantics=("arbitrary",)),
    )(h, *parts, ua, ub, bu1, u2, bu2)


def kernel(h, edge_index, edge_attr, W1, b1, W2, b2, U1, bu1, U2, bu2):
    src = edge_index[0]
    dst = edge_index[1]
    w1a, w1b, w1c = W1[:D], W1[D:2 * D], W1[2 * D:]
    ua, ub = U1[:D], U1[D:]
    b1r = b1.reshape(1, D)
    b2r = b2.reshape(1, D)
    bu1r = bu1.reshape(1, D)
    bu2r = bu2.reshape(1, D)

    a, b = _precompute(h, w1a, w1b)

    es = E // NSPLIT
    parts = []
    for s in range(NSPLIT):
        dst_s = lax.slice_in_dim(dst, s * es, (s + 1) * es)
        src_s = lax.slice_in_dim(src, s * es, (s + 1) * es)
        ea_s = lax.slice_in_dim(edge_attr, s * es, (s + 1) * es)
        g = _make_gather(es)(a, b, dst_s, src_s)
        m2 = _msg(g, ea_s, w1c, b1r, W2, b2r)
        p = _make_scatter(es)(m2, dst_s)
        parts.extend([p[0], p[1]])

    return _update(h, parts, ua, ub, bu1r, U2, bu2r)


# speedup vs baseline: 3.8721x; 1.0380x over previous
import functools

import jax
import jax.numpy as jnp
from jax import lax
from jax.experimental import pallas as pl
from jax.experimental.pallas import tpu as pltpu
from jax.experimental.pallas import tpu_sc as plsc

N, E, D, DE = 10000, 320000, 128, 16
DP = D // 2
NC, NS, L = 2, 16, 16
NW = NC * NS
EW = E // NW
K = 80
KP = K // 2
NCHUNK = EW // K
RPT = 624
REM = N - NS * RPT
ZR = 16

_PREC = lax.Precision.DEFAULT

_mesh = plsc.VectorSubcoreMesh(
    core_axis_name="c", subcore_axis_name="s", num_cores=NC, num_subcores=NS)

_HI = -65536
_LO = 65535
_RND = 32768


def _unpack16(ref, r, g):
    wi = lax.bitcast_convert_type(ref[r, pl.ds(g * L, L)], jnp.int32)
    lo = lax.bitcast_convert_type(wi << 16, jnp.float32)
    hi = lax.bitcast_convert_type(wi & _HI, jnp.float32)
    return lo, hi


def _pack16(lo_val, hi_val):
    lo_i = ((lax.bitcast_convert_type(lo_val, jnp.int32) + _RND) >> 16) & _LO
    hi_i = (lax.bitcast_convert_type(hi_val, jnp.int32) + _RND) & _HI
    return lax.bitcast_convert_type(lo_i | hi_i, jnp.float32)


@functools.partial(
    pl.kernel,
    out_type=jax.ShapeDtypeStruct((E // 2, D), jnp.float32),
    mesh=_mesh,
    scratch_types=[
        pltpu.VMEM((EW,), jnp.int32),
        pltpu.VMEM((EW,), jnp.int32),
        pltpu.VMEM((K, D), jnp.float32),
        pltpu.VMEM((K, D), jnp.float32),
        pltpu.VMEM((K, D), jnp.float32),
        pltpu.VMEM((K, D), jnp.float32),
        pltpu.VMEM((KP, D), jnp.float32),
        pltpu.VMEM((KP, D), jnp.float32),
        pltpu.SemaphoreType.DMA,
        pltpu.SemaphoreType.DMA,
        pltpu.SemaphoreType.DMA,
        pltpu.SemaphoreType.DMA,
    ],
)
def _gather_sum(a_hbm, b_hbm, dst_hbm, src_hbm, out_hbm,
                dstv, srcv, ga0, gb0, ga1, gb1, po0, po1,
                sg0, sg1, so0, so1):
    wid = lax.axis_index("s") * NC + lax.axis_index("c")
    base = wid * EW
    pbase = wid * (EW // 2)

    pltpu.sync_copy(dst_hbm.at[pl.ds(base, EW)], dstv)
    pltpu.sync_copy(src_hbm.at[pl.ds(base, EW)], srcv)

    bufs = ((ga0, gb0, po0, sg0, so0), (ga1, gb1, po1, sg1, so1))

    def gstart(c, b):
        ga, gb, _, sg, _ = bufs[b]
        pltpu.async_copy(a_hbm.at[dstv.at[pl.ds(c * K, K)]], ga, sg)
        pltpu.async_copy(b_hbm.at[srcv.at[pl.ds(c * K, K)]], gb, sg)

    def gwait(b):
        ga, gb, _, sg, _ = bufs[b]
        pltpu.make_async_copy(a_hbm.at[dstv.at[pl.ds(0, K)]], ga, sg).wait()
        pltpu.make_async_copy(b_hbm.at[srcv.at[pl.ds(0, K)]], gb, sg).wait()

    def add(b):
        ga, gb, po, _, _ = bufs[b]

        @pl.loop(0, KP, unroll=2)
        def _pair(t):
            r0 = 2 * t
            r1 = 2 * t + 1
            for g in range(D // L):
                sl = pl.ds(g * L, L)
                s0 = ga[r0, sl] + gb[r0, sl]
                s1 = ga[r1, sl] + gb[r1, sl]
                po[t, sl] = _pack16(s0, s1)

    def ostart(c, b):
        _, _, po, _, so = bufs[b]
        pltpu.async_copy(po, out_hbm.at[pl.ds(pbase + c * KP, KP)], so)

    def owait(b):
        _, _, po, _, so = bufs[b]
        pltpu.make_async_copy(po, out_hbm.at[pl.ds(pbase, KP)], so).wait()

    gstart(0, 0)
    gwait(0)
    gstart(1, 1)
    add(0)
    ostart(0, 0)

    @pl.loop(0, (NCHUNK - 3) // 2)
    def _pair_loop(p):
        c1 = 2 * p + 1
        gwait(1)
        owait(0)
        gstart(c1 + 1, 0)
        add(1)
        ostart(c1, 1)
        c2 = 2 * p + 2
        gwait(0)
        owait(1)
        gstart(c2 + 1, 1)
        add(0)
        ostart(c2, 0)

    gwait(1)
    owait(0)
    gstart(NCHUNK - 1, 0)
    add(1)
    ostart(NCHUNK - 2, 1)
    gwait(0)
    owait(1)
    add(0)
    ostart(NCHUNK - 1, 0)
    owait(0)


@functools.partial(
    pl.kernel,
    out_type=jax.ShapeDtypeStruct((NC, N, D), jnp.float32),
    mesh=_mesh,
    scratch_types=[
        pltpu.VMEM((K,), jnp.int32),
        pltpu.VMEM((K,), jnp.int32),
        pltpu.VMEM((K, D), jnp.float32),
        pltpu.VMEM((K, D), jnp.float32),
        pltpu.VMEM((ZR, D), jnp.float32),
        pltpu.VMEM_SHARED((N, D), jnp.float32),
        pltpu.SemaphoreType.DMA,
        pltpu.SemaphoreType.DMA,
        pltpu.SemaphoreType.DMA,
        pltpu.SemaphoreType.DMA,
    ],
)
def _scatter_add(m2_hbm, dst_hbm, out_hbm,
                 idx0, idx1, mf0, mf1, zb, aggr,
                 sl0, sl1, ss0, ss1):
    cid = lax.axis_index("c")
    sid = lax.axis_index("s")
    wid = sid * NC + cid
    base = wid * EW
    pbase = wid * (EW // 2)

    @pl.loop(0, ZR)
    def _zrow(r):
        for j in range(D // L):
            zb[r, pl.ds(j * L, L)] = jnp.zeros((L,), jnp.float32)

    @pl.loop(0, RPT // ZR)
    def _zcopy(z):
        pltpu.sync_copy(zb, aggr.at[pl.ds(sid * RPT + z * ZR, ZR)])

    @pl.when(sid == NS - 1)
    def _ztail():
        pltpu.sync_copy(zb, aggr.at[pl.ds(NS * RPT, REM)])

    plsc.subcore_barrier()

    bufs = ((idx0, mf0, sl0, ss0), (idx1, mf1, sl1, ss1))

    def lstart(c, b):
        idxb, mf, sl, _ = bufs[b]
        pltpu.async_copy(dst_hbm.at[pl.ds(base + c * K, K)], idxb, sl)
        pltpu.async_copy(m2_hbm.at[pl.ds(base + c * K, K)], mf, sl)

    def lwait(b):
        idxb, mf, sl, _ = bufs[b]
        pltpu.make_async_copy(dst_hbm.at[pl.ds(base, K)], idxb, sl).wait()
        pltpu.make_async_copy(m2_hbm.at[pl.ds(base, K)], mf, sl).wait()

    def sstart(b):
        idxb, mf, _, ss = bufs[b]
        pltpu.async_copy(mf, aggr.at[idxb], ss, add=True)

    def swait(b):
        idxb, mf, _, ss = bufs[b]
        pltpu.make_async_copy(mf, aggr.at[idxb], ss).wait()

    lstart(0, 0)
    lwait(0)
    lstart(1, 1)
    sstart(0)

    @pl.loop(0, (NCHUNK - 3) // 2)
    def _pair_loop(p):
        lwait(1)
        swait(0)
        lstart(2 * p + 2, 0)
        sstart(1)
        lwait(0)
        swait(1)
        lstart(2 * p + 3, 1)
        sstart(0)

    lwait(1)
    swait(0)
    lstart(NCHUNK - 1, 0)
    sstart(1)
    lwait(0)
    swait(1)
    sstart(0)
    swait(0)

    plsc.subcore_barrier()

    pltpu.sync_copy(aggr.at[pl.ds(sid * RPT, RPT)],
                    out_hbm.at[cid].at[pl.ds(sid * RPT, RPT)])

    @pl.when(sid == NS - 1)
    def _wtail():
        pltpu.sync_copy(aggr.at[pl.ds(NS * RPT, REM)],
                        out_hbm.at[cid].at[pl.ds(NS * RPT, REM)])


def _dot(x, w):
    return jnp.dot(x, w, precision=_PREC, preferred_element_type=jnp.float32)


def _pre_body(h_ref, wa_ref, wb_ref, a_ref, b_ref):
    hblk = h_ref[...]
    a_ref[...] = _dot(hblk, wa_ref[...])
    b_ref[...] = _dot(hblk, wb_ref[...])


def _precompute(h, w1a, w1b):
    blk = 1000
    grid = N // blk
    return pl.pallas_call(
        _pre_body,
        grid=(grid,),
        in_specs=[
            pl.BlockSpec((blk, D), lambda i: (i, 0)),
            pl.BlockSpec((D, D), lambda i: (0, 0)),
            pl.BlockSpec((D, D), lambda i: (0, 0)),
        ],
        out_specs=[
            pl.BlockSpec((blk, D), lambda i: (i, 0)),
            pl.BlockSpec((blk, D), lambda i: (i, 0)),
        ],
        out_shape=[
            jax.ShapeDtypeStruct((N, D), jnp.float32),
            jax.ShapeDtypeStruct((N, D), jnp.float32),
        ],
        compiler_params=pltpu.CompilerParams(
            dimension_semantics=("arbitrary",)),
    )(h, w1a, w1b)


def _msg_body(gp_ref, ea_ref, w1c_ref, b1_ref, w2_ref, b2_ref, o_ref):
    g16 = pltpu.bitcast(gp_ref[...], jnp.bfloat16)
    x = g16.astype(jnp.float32) + _dot(ea_ref[...], w1c_ref[...]) + b1_ref[...]
    x = jnp.maximum(x, 0.0)
    y = _dot(x, w2_ref[...]) + b2_ref[...]
    o_ref[...] = jnp.maximum(y, 0.0)


def _msg(gp, ea, w1c, b1, w2, b2):
    blk = 2000
    grid = E // blk
    return pl.pallas_call(
        _msg_body,
        grid=(grid,),
        in_specs=[
            pl.BlockSpec((blk // 2, D), lambda i: (i, 0)),
            pl.BlockSpec((blk, DE), lambda i: (i, 0)),
            pl.BlockSpec((DE, D), lambda i: (0, 0)),
            pl.BlockSpec((1, D), lambda i: (0, 0)),
            pl.BlockSpec((D, D), lambda i: (0, 0)),
            pl.BlockSpec((1, D), lambda i: (0, 0)),
        ],
        out_specs=pl.BlockSpec((blk, D), lambda i: (i, 0)),
        out_shape=jax.ShapeDtypeStruct((E, D), jnp.float32),
        compiler_params=pltpu.CompilerParams(
            dimension_semantics=("arbitrary",)),
    )(gp, ea, w1c, b1, w2, b2)


def _upd_body(h_ref, p0_ref, p1_ref, ua_ref, ub_ref, bu1_ref, u2_ref,
              bu2_ref, o_ref):
    aggr = p0_ref[...] + p1_ref[...]
    u = _dot(h_ref[...], ua_ref[...]) + _dot(aggr, ub_ref[...]) + bu1_ref[...]
    u = jnp.maximum(u, 0.0)
    o_ref[...] = jnp.maximum(_dot(u, u2_ref[...]) + bu2_ref[...], 0.0)


def _update(h, p0, p1, ua, ub, bu1, u2, bu2):
    blk = 1000
    grid = N // blk
    nblk = pl.BlockSpec((blk, D), lambda i: (i, 0))
    wblk = pl.BlockSpec((D, D), lambda i: (0, 0))
    bblk = pl.BlockSpec((1, D), lambda i: (0, 0))
    return pl.pallas_call(
        _upd_body,
        grid=(grid,),
        in_specs=[nblk, nblk, nblk, wblk, wblk, bblk, wblk, bblk],
        out_specs=nblk,
        out_shape=jax.ShapeDtypeStruct((N, D), jnp.float32),
        compiler_params=pltpu.CompilerParams(
            dimension_semantics=("arbitrary",)),
    )(h, p0, p1, ua, ub, bu1, u2, bu2)


def kernel(h, edge_index, edge_attr, W1, b1, W2, b2, U1, bu1, U2, bu2):
    src = edge_index[0]
    dst = edge_index[1]
    w1a, w1b, w1c = W1[:D], W1[D:2 * D], W1[2 * D:]
    ua, ub = U1[:D], U1[D:]
    b1r = b1.reshape(1, D)
    b2r = b2.reshape(1, D)
    bu1r = bu1.reshape(1, D)
    bu2r = bu2.reshape(1, D)

    a, b = _precompute(h, w1a, w1b)

    gp = _gather_sum(a, b, dst, src)
    m2p = _msg(gp, edge_attr, w1c, b1r, W2, b2r)
    parts = _scatter_add(m2p, dst)
    return _update(h, parts[0], parts[1], ua, ub, bu1r, U2, bu2r)
